# baseline (speedup 1.0000x reference)
import jax
import jax.numpy as jnp
from jax.experimental import pallas as pl

N = 512
T = 12
B = 8
D_IN = 2
U = 64
NM = 3
NCLS = 5


def _mm(a, b):
    return jax.lax.dot_general(
        a, b, (((1,), (0,)), ((), ())), preferred_element_type=jnp.float32
    )


def _dcgru_kernel(s_ref, xin_ref, xin_bn_ref, wg0_ref, wg1_ref, wg2_ref,
                  bg_ref, wc0_ref, wc1_ref, wc2_ref, bc_ref, fcw_ref,
                  fcb_ref, out_ref):
    Sb = s_ref[...].astype(jnp.bfloat16)
    bg = bg_ref[...]
    bc = bc_ref[...]
    bf = jnp.bfloat16

    def nb_to_bn_h(Dnb):
        return jnp.concatenate([Dnb[:, b * U:(b + 1) * U] for b in range(B)],
                               axis=0)

    def nb_to_bn_in(Dnb):
        return jnp.concatenate(
            [Dnb[:, N + b * D_IN:N + (b + 1) * D_IN] for b in range(B)],
            axis=0)

    def bn_to_nb(Xbn):
        return jnp.concatenate([Xbn[b * N:(b + 1) * N, :] for b in range(B)],
                               axis=1)

    def step(t, carry):
        h_nb, h_bn = carry
        xin_t = xin_ref[t]
        X0 = jnp.concatenate([h_nb, xin_t], axis=1)
        X1 = _mm(Sb, X0).astype(bf)
        X2 = (2.0 * _mm(Sb, X1)).astype(bf) - X0

        in0 = xin_bn_ref[t]
        in1 = nb_to_bn_in(X1)
        in2 = nb_to_bn_in(X2)
        h_bn_b = h_bn.astype(bf)
        Xbn0 = jnp.concatenate([h_bn_b, in0], axis=1)
        Xbn1 = jnp.concatenate([nb_to_bn_h(X1), in1], axis=1)
        Xbn2 = jnp.concatenate([nb_to_bn_h(X2), in2], axis=1)

        gates = (bg + _mm(Xbn0, wg0_ref[...])
                 + _mm(Xbn1, wg1_ref[...])
                 + _mm(Xbn2, wg2_ref[...]))
        gates = jax.nn.sigmoid(gates)
        r = gates[:, :U]
        u = gates[:, U:]

        rh = (r * h_bn).astype(bf)
        R0 = bn_to_nb(rh)
        R1 = _mm(Sb, R0).astype(bf)
        R2 = (2.0 * _mm(Sb, R1)).astype(bf) - R0

        Ybn0 = jnp.concatenate([rh, in0], axis=1)
        Ybn1 = jnp.concatenate([nb_to_bn_h(R1), in1], axis=1)
        Ybn2 = jnp.concatenate([nb_to_bn_h(R2), in2], axis=1)

        c = (bc + _mm(Ybn0, wc0_ref[...])
             + _mm(Ybn1, wc1_ref[...])
             + _mm(Ybn2, wc2_ref[...]))
        c = jnp.tanh(c)

        h_new_bn = u * h_bn + (1.0 - u) * c
        h_new_nb = bn_to_nb(h_new_bn.astype(bf))
        return (h_new_nb, h_new_bn)

    h0_nb = jnp.zeros((N, B * U), jnp.bfloat16)
    h0_bn = jnp.zeros((B * N, U), jnp.float32)
    _, h_bn = jax.lax.fori_loop(0, T, step, (h0_nb, h0_bn))

    h = jnp.maximum(h_bn, 0.0)
    logits = _mm(h, fcw_ref[...]) + fcb_ref[...]
    for b in range(B):
        out_ref[b:b + 1, :] = jnp.max(logits[b * N:(b + 1) * N, :], axis=0,
                                      keepdims=True)


@jax.jit
def kernel(input_seq, seq_lengths, supports, Wg0, bg0, Wc0, bc0, fc_w, fc_b):
    del seq_lengths
    S = supports[0]
    xin = jnp.transpose(input_seq, (1, 2, 0, 3)).reshape(T, N, B * D_IN)
    xin = xin.astype(jnp.bfloat16)
    xin_bn = jnp.transpose(input_seq, (1, 0, 2, 3)).reshape(T, B * N, D_IN)
    xin_bn = xin_bn.astype(jnp.bfloat16)
    Wgr = Wg0.reshape(D_IN + U, NM, 2 * U)
    Wg = jnp.concatenate([Wgr[D_IN:], Wgr[:D_IN]], axis=0)
    Wg = Wg.astype(jnp.bfloat16)
    Wcr = Wc0.reshape(D_IN + U, NM, U)
    Wc = jnp.concatenate([Wcr[D_IN:], Wcr[:D_IN]], axis=0)
    Wc = Wc.astype(jnp.bfloat16)

    out = pl.pallas_call(
        _dcgru_kernel,
        out_shape=jax.ShapeDtypeStruct((B, NCLS), jnp.float32),
    )(S, xin, xin_bn, Wg[:, 0], Wg[:, 1], Wg[:, 2], bg0[None],
      Wc[:, 0], Wc[:, 1], Wc[:, 2], bc0[None], fc_w, fc_b[None])
    return out


# speedup vs baseline: 1.0975x; 1.0975x over previous
import jax
import jax.numpy as jnp
from jax.experimental import pallas as pl

N = 512
T = 12
B = 8
D_IN = 2
U = 64
NM = 3
NCLS = 5


def _mm(a, b):
    return jax.lax.dot_general(
        a, b, (((1,), (0,)), ((), ())), preferred_element_type=jnp.float32
    )


def _dcgru_kernel(s_ref, xin_ref, wg0_ref, wg1_ref, wg2_ref,
                  bg_ref, wc0_ref, wc1_ref, wc2_ref, bc_ref, fcw_ref,
                  fcb_ref, out_ref):
    Sb = s_ref[...].astype(jnp.bfloat16)
    bg = bg_ref[...]
    bc = bc_ref[...]
    bf = jnp.bfloat16

    def nb_to_bn_h(Dnb):
        return jnp.concatenate([Dnb[:, b * U:(b + 1) * U] for b in range(B)],
                               axis=0)

    def nb_to_bn_in(Dnb):
        return jnp.concatenate(
            [Dnb[:, N + b * D_IN:N + (b + 1) * D_IN] for b in range(B)],
            axis=0)

    def bn_to_nb(Xbn):
        return jnp.concatenate([Xbn[b * N:(b + 1) * N, :] for b in range(B)],
                               axis=1)

    def step(t, carry):
        h_nb, h_bn = carry
        xin_t = xin_ref[t]
        X0 = jnp.concatenate([h_nb, xin_t], axis=1)
        X1 = _mm(Sb, X0).astype(bf)
        X2 = (2.0 * _mm(Sb, X1) - X0.astype(jnp.float32)).astype(bf)

        in0 = nb_to_bn_in(X0)
        in1 = nb_to_bn_in(X1)
        in2 = nb_to_bn_in(X2)
        h_bn_b = h_bn.astype(bf)
        Xbn0 = jnp.concatenate([h_bn_b, in0], axis=1)
        Xbn1 = jnp.concatenate([nb_to_bn_h(X1), in1], axis=1)
        Xbn2 = jnp.concatenate([nb_to_bn_h(X2), in2], axis=1)

        gates = (bg + _mm(Xbn0, wg0_ref[...])
                 + _mm(Xbn1, wg1_ref[...])
                 + _mm(Xbn2, wg2_ref[...]))
        gates = jax.nn.sigmoid(gates)
        r = gates[:, :U]
        u = gates[:, U:]

        rh = (r * h_bn).astype(bf)
        R0 = bn_to_nb(rh)
        R1 = _mm(Sb, R0).astype(bf)
        R2 = (2.0 * _mm(Sb, R1) - R0.astype(jnp.float32)).astype(bf)

        Ybn0 = jnp.concatenate([rh, in0], axis=1)
        Ybn1 = jnp.concatenate([nb_to_bn_h(R1), in1], axis=1)
        Ybn2 = jnp.concatenate([nb_to_bn_h(R2), in2], axis=1)

        c = (bc + _mm(Ybn0, wc0_ref[...])
             + _mm(Ybn1, wc1_ref[...])
             + _mm(Ybn2, wc2_ref[...]))
        c = jnp.tanh(c)

        h_new_bn = u * h_bn + (1.0 - u) * c
        h_new_nb = bn_to_nb(h_new_bn.astype(bf))
        return (h_new_nb, h_new_bn)

    h0_nb = jnp.zeros((N, B * U), jnp.bfloat16)
    h0_bn = jnp.zeros((B * N, U), jnp.float32)
    _, h_bn = jax.lax.fori_loop(0, T, step, (h0_nb, h0_bn))

    h = jnp.maximum(h_bn, 0.0)
    logits = _mm(h, fcw_ref[...]) + fcb_ref[...]
    for b in range(B):
        out_ref[b:b + 1, :] = jnp.max(logits[b * N:(b + 1) * N, :], axis=0,
                                      keepdims=True)


@jax.jit
def kernel(input_seq, seq_lengths, supports, Wg0, bg0, Wc0, bc0, fc_w, fc_b):
    del seq_lengths
    S = supports[0]
    xin = jnp.transpose(input_seq, (1, 2, 0, 3)).reshape(T, N, B * D_IN)
    xin = xin.astype(jnp.bfloat16)
    Wgr = Wg0.reshape(D_IN + U, NM, 2 * U)
    Wg = jnp.concatenate([Wgr[D_IN:], Wgr[:D_IN]], axis=0)
    Wg = Wg.astype(jnp.bfloat16)
    Wcr = Wc0.reshape(D_IN + U, NM, U)
    Wc = jnp.concatenate([Wcr[D_IN:], Wcr[:D_IN]], axis=0)
    Wc = Wc.astype(jnp.bfloat16)

    out = pl.pallas_call(
        _dcgru_kernel,
        out_shape=jax.ShapeDtypeStruct((B, NCLS), jnp.float32),
    )(S, xin, Wg[:, 0], Wg[:, 1], Wg[:, 2], bg0[None],
      Wc[:, 0], Wc[:, 1], Wc[:, 2], bc0[None], fc_w, fc_b[None])
    return out


# speedup vs baseline: 1.1433x; 1.0418x over previous
import jax
import jax.numpy as jnp
from jax.experimental import pallas as pl

N = 512
T = 12
B = 8
D_IN = 2
U = 64
NM = 3
NCLS = 5


def _mm(a, b):
    return jax.lax.dot_general(
        a, b, (((1,), (0,)), ((), ())), preferred_element_type=jnp.float32
    )


def _dcgru_kernel(s_ref, xin_ref, wg0_ref, wg1_ref, wg2_ref,
                  bg_ref, wc0_ref, wc1_ref, wc2_ref, bc_ref, fcw_ref,
                  fcb_ref, out_ref):
    Sb = s_ref[...].astype(jnp.bfloat16)
    bg = bg_ref[...]
    bc = bc_ref[...]
    bf = jnp.bfloat16

    def nb_to_bn_h(Dnb):
        return jnp.concatenate([Dnb[:, b * U:(b + 1) * U] for b in range(B)],
                               axis=0)

    def nb_to_bn_in(Dnb):
        return jnp.concatenate(
            [Dnb[:, N + b * D_IN:N + (b + 1) * D_IN] for b in range(B)],
            axis=0)

    def bn_to_nb(Xbn):
        return jnp.concatenate([Xbn[b * N:(b + 1) * N, :] for b in range(B)],
                               axis=1)

    def step(t, carry):
        h_nb, h_bn = carry
        xin_t = xin_ref[t]
        X0 = jnp.concatenate([h_nb, xin_t], axis=1)
        X1 = _mm(Sb, X0).astype(bf)
        X2 = (2.0 * _mm(Sb, X1) - X0.astype(jnp.float32)).astype(bf)

        in0 = nb_to_bn_in(X0)
        in1 = nb_to_bn_in(X1)
        in2 = nb_to_bn_in(X2)
        h_bn_b = h_bn.astype(bf)
        Xbn0 = jnp.concatenate([h_bn_b, in0], axis=1)
        Xbn1 = jnp.concatenate([nb_to_bn_h(X1), in1], axis=1)
        Xbn2 = jnp.concatenate([nb_to_bn_h(X2), in2], axis=1)

        gates = (bg + _mm(Xbn0, wg0_ref[...])
                 + _mm(Xbn1, wg1_ref[...])
                 + _mm(Xbn2, wg2_ref[...]))
        gates = jax.nn.sigmoid(gates)
        r = gates[:, :U]
        u = gates[:, U:]

        rh = (r * h_bn).astype(bf)
        R0 = bn_to_nb(rh)
        R1 = _mm(Sb, R0).astype(bf)
        R2 = (2.0 * _mm(Sb, R1) - R0.astype(jnp.float32)).astype(bf)

        Ybn0 = jnp.concatenate([rh, in0], axis=1)
        Ybn1 = jnp.concatenate([nb_to_bn_h(R1), in1], axis=1)
        Ybn2 = jnp.concatenate([nb_to_bn_h(R2), in2], axis=1)

        c = (bc + _mm(Ybn0, wc0_ref[...])
             + _mm(Ybn1, wc1_ref[...])
             + _mm(Ybn2, wc2_ref[...]))
        c = jnp.tanh(c)

        h_new_bn = u * h_bn + (1.0 - u) * c
        h_new_nb = bn_to_nb(h_new_bn.astype(bf))
        return (h_new_nb, h_new_bn)

    Xi0 = xin_ref[0]
    Xi1 = _mm(Sb, Xi0).astype(bf)
    Xi2 = (2.0 * _mm(Sb, Xi1) - Xi0.astype(jnp.float32)).astype(bf)

    def in_bn(Dn):
        return jnp.concatenate(
            [Dn[:, b * D_IN:(b + 1) * D_IN] for b in range(B)], axis=0)

    i0, i1, i2 = in_bn(Xi0), in_bn(Xi1), in_bn(Xi2)
    g0 = (bg + _mm(i0, wg0_ref[U:, :]) + _mm(i1, wg1_ref[U:, :])
          + _mm(i2, wg2_ref[U:, :]))
    u0 = jax.nn.sigmoid(g0[:, U:])
    c0 = (bc + _mm(i0, wc0_ref[U:, :]) + _mm(i1, wc1_ref[U:, :])
          + _mm(i2, wc2_ref[U:, :]))
    c0 = jnp.tanh(c0)
    h1_bn = (1.0 - u0) * c0
    h1_nb = bn_to_nb(h1_bn.astype(bf))

    _, h_bn = jax.lax.fori_loop(1, T, step, (h1_nb, h1_bn))

    h = jnp.maximum(h_bn, 0.0)
    logits = _mm(h, fcw_ref[...]) + fcb_ref[...]
    for b in range(B):
        out_ref[b:b + 1, :] = jnp.max(logits[b * N:(b + 1) * N, :], axis=0,
                                      keepdims=True)


@jax.jit
def kernel(input_seq, seq_lengths, supports, Wg0, bg0, Wc0, bc0, fc_w, fc_b):
    del seq_lengths
    S = supports[0]
    xin = jnp.transpose(input_seq, (1, 2, 0, 3)).reshape(T, N, B * D_IN)
    xin = xin.astype(jnp.bfloat16)
    Wgr = Wg0.reshape(D_IN + U, NM, 2 * U)
    Wg = jnp.concatenate([Wgr[D_IN:], Wgr[:D_IN]], axis=0)
    Wg = Wg.astype(jnp.bfloat16)
    Wcr = Wc0.reshape(D_IN + U, NM, U)
    Wc = jnp.concatenate([Wcr[D_IN:], Wcr[:D_IN]], axis=0)
    Wc = Wc.astype(jnp.bfloat16)

    out = pl.pallas_call(
        _dcgru_kernel,
        out_shape=jax.ShapeDtypeStruct((B, NCLS), jnp.float32),
    )(S, xin, Wg[:, 0], Wg[:, 1], Wg[:, 2], bg0[None],
      Wc[:, 0], Wc[:, 1], Wc[:, 2], bc0[None], fc_w, fc_b[None])
    return out


# speedup vs baseline: 1.1436x; 1.0003x over previous
import jax
import jax.numpy as jnp
from jax.experimental import pallas as pl

N = 512
T = 12
B = 8
D_IN = 2
U = 64
NM = 3
NCLS = 5


def _mm(a, b):
    return jax.lax.dot_general(
        a, b, (((1,), (0,)), ((), ())), preferred_element_type=jnp.float32
    )


def _dcgru_kernel(s_ref, xin_ref, wg0_ref, wg1_ref, wg2_ref,
                  bg_ref, wc0_ref, wc1_ref, wc2_ref, bc_ref, fcw_ref,
                  fcb_ref, out_ref):
    Sb = s_ref[...].astype(jnp.bfloat16)
    bg = bg_ref[...]
    bc = bc_ref[...]
    bf = jnp.bfloat16

    def nb_to_bn_h(Dnb):
        return jnp.concatenate([Dnb[:, b * U:(b + 1) * U] for b in range(B)],
                               axis=0)

    def nb_to_bn_in(Dnb):
        return jnp.concatenate(
            [Dnb[:, N + b * D_IN:N + (b + 1) * D_IN] for b in range(B)],
            axis=0)

    def bn_to_nb(Xbn):
        return jnp.concatenate([Xbn[b * N:(b + 1) * N, :] for b in range(B)],
                               axis=1)

    def step(t, carry):
        h_nb, h_bn = carry
        xin_t = xin_ref[t]
        X0 = jnp.concatenate([h_nb, xin_t], axis=1)
        X1 = _mm(Sb, X0).astype(bf)
        X2 = (2.0 * _mm(Sb, X1) - X0.astype(jnp.float32)).astype(bf)

        in0 = nb_to_bn_in(X0)
        in1 = nb_to_bn_in(X1)
        in2 = nb_to_bn_in(X2)
        h_bn_b = h_bn.astype(bf)
        Xbn0 = jnp.concatenate([h_bn_b, in0], axis=1)
        Xbn1 = jnp.concatenate([nb_to_bn_h(X1), in1], axis=1)
        Xbn2 = jnp.concatenate([nb_to_bn_h(X2), in2], axis=1)

        gates = (bg + _mm(Xbn0, wg0_ref[...])
                 + _mm(Xbn1, wg1_ref[...])
                 + _mm(Xbn2, wg2_ref[...]))
        gates = jax.nn.sigmoid(gates)
        r = gates[:, :U]
        u = gates[:, U:]

        rh = (r * h_bn).astype(bf)
        R0 = bn_to_nb(rh)
        R1 = _mm(Sb, R0).astype(bf)
        R2 = (2.0 * _mm(Sb, R1) - R0.astype(jnp.float32)).astype(bf)

        Ybn0 = jnp.concatenate([rh, in0], axis=1)
        Ybn1 = jnp.concatenate([nb_to_bn_h(R1), in1], axis=1)
        Ybn2 = jnp.concatenate([nb_to_bn_h(R2), in2], axis=1)

        c = (bc + _mm(Ybn0, wc0_ref[...])
             + _mm(Ybn1, wc1_ref[...])
             + _mm(Ybn2, wc2_ref[...]))
        c = jnp.tanh(c)

        h_new_bn = u * h_bn + (1.0 - u) * c
        h_new_nb = bn_to_nb(h_new_bn.astype(bf))
        return (h_new_nb, h_new_bn)

    Xi0 = xin_ref[0]
    Xi1 = _mm(Sb, Xi0).astype(bf)
    Xi2 = (2.0 * _mm(Sb, Xi1) - Xi0.astype(jnp.float32)).astype(bf)

    def in_bn(Dn):
        return jnp.concatenate(
            [Dn[:, b * D_IN:(b + 1) * D_IN] for b in range(B)], axis=0)

    i0, i1, i2 = in_bn(Xi0), in_bn(Xi1), in_bn(Xi2)
    g0 = (bg + _mm(i0, wg0_ref[U:, :]) + _mm(i1, wg1_ref[U:, :])
          + _mm(i2, wg2_ref[U:, :]))
    u0 = jax.nn.sigmoid(g0[:, U:])
    c0 = (bc + _mm(i0, wc0_ref[U:, :]) + _mm(i1, wc1_ref[U:, :])
          + _mm(i2, wc2_ref[U:, :]))
    c0 = jnp.tanh(c0)
    h1_bn = (1.0 - u0) * c0
    h1_nb = bn_to_nb(h1_bn.astype(bf))

    _, h_bn = jax.lax.fori_loop(1, T, step, (h1_nb, h1_bn), unroll=2)

    h = jnp.maximum(h_bn, 0.0)
    logits = _mm(h, fcw_ref[...]) + fcb_ref[...]
    for b in range(B):
        out_ref[b:b + 1, :] = jnp.max(logits[b * N:(b + 1) * N, :], axis=0,
                                      keepdims=True)


@jax.jit
def kernel(input_seq, seq_lengths, supports, Wg0, bg0, Wc0, bc0, fc_w, fc_b):
    del seq_lengths
    S = supports[0]
    xin = jnp.transpose(input_seq, (1, 2, 0, 3)).reshape(T, N, B * D_IN)
    xin = xin.astype(jnp.bfloat16)
    Wgr = Wg0.reshape(D_IN + U, NM, 2 * U)
    Wg = jnp.concatenate([Wgr[D_IN:], Wgr[:D_IN]], axis=0)
    Wg = Wg.astype(jnp.bfloat16)
    Wcr = Wc0.reshape(D_IN + U, NM, U)
    Wc = jnp.concatenate([Wcr[D_IN:], Wcr[:D_IN]], axis=0)
    Wc = Wc.astype(jnp.bfloat16)

    out = pl.pallas_call(
        _dcgru_kernel,
        out_shape=jax.ShapeDtypeStruct((B, NCLS), jnp.float32),
    )(S, xin, Wg[:, 0], Wg[:, 1], Wg[:, 2], bg0[None],
      Wc[:, 0], Wc[:, 1], Wc[:, 2], bc0[None], fc_w, fc_b[None])
    return out
